# Initial kernel scaffold; baseline (speedup 1.0000x reference)
import jax
import jax.numpy as jnp
from jax.experimental import pallas as pl


def kernel(logits, scores, finished_flag, position):
    raise NotImplementedError("write your pallas kernel here")


# speedup vs baseline: 1.9490x; 1.9490x over previous
import functools

import jax
import jax.numpy as jnp
from jax import lax
from jax.experimental import pallas as pl
from jax.experimental.pallas import tpu as pltpu

BEAM = 8
PAD, UNK, BOS, EOS = 1, 3, 0, 2
NEG_INF = float("-inf")


def _topk_body(x_ref, sc_ref, fin_ref, ts_ref, wi_ref, bi_ref):
    x = x_ref[0]
    sc_row = sc_ref[0, 0]
    fin_row = fin_ref[0, 0]
    beams, vocab = x.shape
    m = jnp.max(x, axis=1, keepdims=True)
    e = jnp.sum(jnp.exp(x - m), axis=1, keepdims=True)
    c = sc_row[:, None] - m - jnp.log(e)

    col = lax.broadcasted_iota(jnp.int32, (beams, vocab), 1)
    row = lax.broadcasted_iota(jnp.int32, (beams, vocab), 0)
    flat = row * vocab + col

    vals = x + c
    special = (col == PAD) | (col == UNK) | (col == BOS)
    vals = jnp.where(special, NEG_INF, vals)
    fin = fin_row[:, None] != 0
    vals = jnp.where(fin & (col == PAD), jnp.inf, vals)

    lane = lax.broadcasted_iota(jnp.int32, (BEAM,), 0)
    ts_acc = jnp.zeros((BEAM,), jnp.float32)
    wi_acc = jnp.zeros((BEAM,), jnp.int32)
    bi_acc = jnp.zeros((BEAM,), jnp.int32)
    for k in range(BEAM):
        best = jnp.max(vals)
        idx = jnp.min(jnp.where(vals == best, flat, jnp.int32(2**31 - 1)))
        ts_acc = jnp.where(lane == k, best, ts_acc)
        wi_acc = jnp.where(lane == k, idx % vocab, wi_acc)
        bi_acc = jnp.where(lane == k, idx // vocab, bi_acc)
        vals = jnp.where(flat == idx, NEG_INF, vals)
    ts_ref[0, 0] = ts_acc
    wi_ref[0, 0] = wi_acc
    bi_ref[0, 0] = bi_acc


@functools.partial(jax.jit, static_argnames=())
def _run(x, sc_sel, fin):
    batch = x.shape[0] // BEAM
    vocab = x.shape[-1]
    xb = x.reshape(batch, BEAM, vocab)
    grid = (batch,)
    out = pl.pallas_call(
        _topk_body,
        grid=grid,
        in_specs=[
            pl.BlockSpec((1, BEAM, vocab), lambda i: (i, 0, 0)),
            pl.BlockSpec((1, 1, BEAM), lambda i: (i, 0, 0)),
            pl.BlockSpec((1, 1, BEAM), lambda i: (i, 0, 0)),
        ],
        out_specs=[
            pl.BlockSpec((1, 1, BEAM), lambda i: (i, 0, 0)),
            pl.BlockSpec((1, 1, BEAM), lambda i: (i, 0, 0)),
            pl.BlockSpec((1, 1, BEAM), lambda i: (i, 0, 0)),
        ],
        out_shape=[
            jax.ShapeDtypeStruct((batch, 1, BEAM), jnp.float32),
            jax.ShapeDtypeStruct((batch, 1, BEAM), jnp.int32),
            jax.ShapeDtypeStruct((batch, 1, BEAM), jnp.int32),
        ],
    )(xb, sc_sel.reshape(batch, 1, BEAM), fin.reshape(batch, 1, BEAM))
    return tuple(o.reshape(batch, BEAM) for o in out)


def kernel(logits, scores, finished_flag, position):
    x = logits[:, -1, :]
    sc_sel = lax.dynamic_index_in_dim(scores, position - 1, axis=2, keepdims=False)
    fin = finished_flag.astype(jnp.int32)
    ts, wi, bi = _run(x, sc_sel.reshape(-1), fin)
    return ts, wi, bi
